# baseline (speedup 1.0000x reference)
import functools

import jax
import jax.numpy as jnp
from jax import lax
from jax.experimental import pallas as pl
from jax.experimental.pallas import tpu as pltpu
from jax.experimental.pallas import tpu_sc as plsc

NC, NS, L = 2, 16, 16
NW = NC * NS
B = 16384
D = 64
BPW = B // NW
CW = 128
NCHUNK = BPW // CW

_LOG1P_C = (
    0.999998763883492,
    -0.4998719252774506,
    0.33112058367396247,
    -0.2351488240931608,
    0.14943483645817232,
    -0.06658820573216659,
    0.01420285926015165,
)


def _softplus(x):
    z = jnp.exp(-jnp.abs(x))
    p = jnp.float32(_LOG1P_C[-1])
    for c in _LOG1P_C[-2::-1]:
        p = p * z + jnp.float32(c)
    return jnp.maximum(x, jnp.float32(0.0)) + p * z


def _irt_body(sid_hbm, qid_hbm, theta_hbm, alpha_hbm, beta_hbm, out_hbm,
              sid_v, qid_v, theta_v, alpha_v, beta_v, out_v, sem):
    wid = lax.axis_index("s") * NC + lax.axis_index("c")

    pltpu.sync_copy(sid_hbm.at[pl.ds(wid * NCHUNK, NCHUNK)], sid_v)
    pltpu.sync_copy(qid_hbm.at[pl.ds(wid * NCHUNK, NCHUNK)], qid_v)

    copies = []
    for j in range(NCHUNK):
        copies.append(pltpu.async_copy(
            theta_hbm.at[sid_v.at[j]], theta_v.at[pl.ds(j * CW, CW)], sem))
        copies.append(pltpu.async_copy(
            alpha_hbm.at[qid_v.at[j]], alpha_v.at[pl.ds(j * CW, CW)], sem))
        copies.append(pltpu.async_copy(
            beta_hbm.at[qid_v.at[j]], beta_v.at[pl.ds(j * CW, CW)], sem))
    for c in copies:
        c.wait()

    iota = lax.iota(jnp.int32, L)

    def group(g, _):
        rows = iota + g * L

        def dcol(d, acc):
            cols = jnp.full((L,), d, jnp.int32)
            th = plsc.load_gather(theta_v, [rows, cols])
            al = plsc.load_gather(alpha_v, [rows, cols])
            return acc + _softplus(al) * th

        acc = lax.fori_loop(0, D, dcol, jnp.zeros((L,), jnp.float32))
        logit = acc + plsc.load_gather(beta_v, [rows])
        pred = jnp.float32(1.0) / (jnp.float32(1.0) + jnp.exp(-logit))
        plsc.store_scatter(out_v, [rows], pred)
        return 0

    lax.fori_loop(0, BPW // L, group, 0)
    pltpu.sync_copy(out_v, out_hbm.at[pl.ds(wid * BPW, BPW)])


@jax.jit
def _irt(sid, qid, theta_w, alpha_w, beta_w):
    mesh = plsc.VectorSubcoreMesh(
        core_axis_name="c", subcore_axis_name="s",
        num_cores=NC, num_subcores=NS)
    run = pl.kernel(
        _irt_body,
        out_type=jax.ShapeDtypeStruct((B,), jnp.float32),
        mesh=mesh,
        compiler_params=pltpu.CompilerParams(
            needs_layout_passes=False, use_tc_tiling_on_sc=False),
        scratch_types=[
            pltpu.VMEM((NCHUNK, CW), jnp.int32),
            pltpu.VMEM((NCHUNK, CW), jnp.int32),
            pltpu.VMEM((BPW, D), jnp.float32),
            pltpu.VMEM((BPW, D), jnp.float32),
            pltpu.VMEM((BPW,), jnp.float32),
            pltpu.VMEM((BPW,), jnp.float32),
            pltpu.SemaphoreType.DMA,
        ],
    )
    return run(sid, qid, theta_w, alpha_w, beta_w)


def kernel(student_ids, question_ids, theta_w, alpha_w, beta_w):
    sid = student_ids.astype(jnp.int32).reshape(B // CW, CW)
    qid = question_ids.astype(jnp.int32).reshape(B // CW, CW)
    out = _irt(sid, qid, theta_w, alpha_w, beta_w.reshape(-1))
    return out.reshape(B, 1)


# speedup vs baseline: 1.0285x; 1.0285x over previous
import functools

import jax
import jax.numpy as jnp
from jax import lax
from jax.experimental import pallas as pl
from jax.experimental.pallas import tpu as pltpu
from jax.experimental.pallas import tpu_sc as plsc

NC, NS, L = 2, 16, 16
NW = NC * NS
B = 16384
D = 64
BPW = B // NW
CW = 128
NCHUNK = BPW // CW

_LOG1P_C = (
    0.999998763883492,
    -0.4998719252774506,
    0.33112058367396247,
    -0.2351488240931608,
    0.14943483645817232,
    -0.06658820573216659,
    0.01420285926015165,
)


def _softplus(x):
    z = jnp.exp(-jnp.abs(x))
    p = jnp.float32(_LOG1P_C[-1])
    for c in _LOG1P_C[-2::-1]:
        p = p * z + jnp.float32(c)
    return jnp.maximum(x, jnp.float32(0.0)) + p * z


def _irt_body(sid_hbm, qid_hbm, theta_hbm, alpha_hbm, beta_hbm, out_hbm,
              sid_v, qid_v, theta_v, alpha_v, beta_v, out_v, sem):
    wid = lax.axis_index("s") * NC + lax.axis_index("c")

    pltpu.sync_copy(sid_hbm.at[pl.ds(wid * NCHUNK, NCHUNK)], sid_v)
    pltpu.sync_copy(qid_hbm.at[pl.ds(wid * NCHUNK, NCHUNK)], qid_v)

    copies = []
    for j in range(NCHUNK):
        copies.append(pltpu.async_copy(
            theta_hbm.at[sid_v.at[j]], theta_v.at[pl.ds(j * CW, CW)], sem))
        copies.append(pltpu.async_copy(
            alpha_hbm.at[qid_v.at[j]], alpha_v.at[pl.ds(j * CW, CW)], sem))
        copies.append(pltpu.async_copy(
            beta_hbm.at[qid_v.at[j]], beta_v.at[pl.ds(j * CW, CW)], sem))
    for c in copies:
        c.wait()

    iota = lax.iota(jnp.int32, L)

    def group(g, _):
        rows = iota + g * L

        acc = jnp.zeros((L,), jnp.float32)
        for d in range(D):
            cols = jnp.full((L,), d, jnp.int32)
            th = plsc.load_gather(theta_v, [rows, cols])
            al = plsc.load_gather(alpha_v, [rows, cols])
            acc = acc + _softplus(al) * th
        logit = acc + plsc.load_gather(beta_v, [rows])
        pred = jnp.float32(1.0) / (jnp.float32(1.0) + jnp.exp(-logit))
        plsc.store_scatter(out_v, [rows], pred)
        return 0

    lax.fori_loop(0, BPW // L, group, 0)
    pltpu.sync_copy(out_v, out_hbm.at[pl.ds(wid * BPW, BPW)])


@jax.jit
def _irt(sid, qid, theta_w, alpha_w, beta_w):
    mesh = plsc.VectorSubcoreMesh(
        core_axis_name="c", subcore_axis_name="s",
        num_cores=NC, num_subcores=NS)
    run = pl.kernel(
        _irt_body,
        out_type=jax.ShapeDtypeStruct((B,), jnp.float32),
        mesh=mesh,
        compiler_params=pltpu.CompilerParams(
            needs_layout_passes=False, use_tc_tiling_on_sc=False),
        scratch_types=[
            pltpu.VMEM((NCHUNK, CW), jnp.int32),
            pltpu.VMEM((NCHUNK, CW), jnp.int32),
            pltpu.VMEM((BPW, D), jnp.float32),
            pltpu.VMEM((BPW, D), jnp.float32),
            pltpu.VMEM((BPW,), jnp.float32),
            pltpu.VMEM((BPW,), jnp.float32),
            pltpu.SemaphoreType.DMA,
        ],
    )
    return run(sid, qid, theta_w, alpha_w, beta_w)


def kernel(student_ids, question_ids, theta_w, alpha_w, beta_w):
    sid = student_ids.astype(jnp.int32).reshape(B // CW, CW)
    qid = question_ids.astype(jnp.int32).reshape(B // CW, CW)
    out = _irt(sid, qid, theta_w, alpha_w, beta_w.reshape(-1))
    return out.reshape(B, 1)
